# baseline (speedup 1.0000x reference)
import functools

import jax
import jax.numpy as jnp
from jax import lax
from jax.experimental import pallas as pl
from jax.experimental.pallas import tpu as pltpu
from jax.experimental.pallas import tpu_sc as plsc

NUM_ELEMENTS = 1000000
EMBED_DIM = 20
OUT_DIM = 2


@functools.partial(
    pl.kernel,
    out_type=jax.ShapeDtypeStruct((1, 16), jnp.float32),
    mesh=plsc.VectorSubcoreMesh(core_axis_name="c", subcore_axis_name="s"),
    scratch_types=[
        pltpu.VMEM((16,), jnp.int32),
        pltpu.VMEM((16,), jnp.int32),
        pltpu.VMEM((32,), jnp.int32),
        pltpu.VMEM((32,), jnp.float32),
        pltpu.VMEM((2, 32), jnp.float32),
        pltpu.VMEM((16,), jnp.float32),
        pltpu.VMEM((32,), jnp.float32),
        pltpu.VMEM((32,), jnp.int32),
        pltpu.VMEM((16,), jnp.float32),
        pltpu.VMEM((1, 16), jnp.float32),
        pltpu.VMEM_SHARED((16,), jnp.float32),
        pltpu.SemaphoreType.DMA,
    ],
    compiler_params=pltpu.CompilerParams(use_tc_tiling_on_sc=False),
)
def _sc_lookup_logsoftmax(idx_hbm, tflat_hbm, wp_hbm, bp_hbm, out_hbm,
                          zidx_v, idxb_v, fidx_v, rowg_v, w_v, b_v,
                          sprod_v, sidx_v, tvec_v, out_v, shared, sem):
    cid = lax.axis_index("c")
    sid = lax.axis_index("s")

    @pl.when(jnp.logical_and(cid == 0, sid == 0))
    def _():
        io = lax.iota(jnp.int32, 16)
        fzero = jnp.zeros((16,), jnp.float32)

        zidx_v[...] = io * 0
        idx_bcast = pltpu.async_copy(idx_hbm.at[zidx_v], idxb_v, sem)

        tvec_v[...] = fzero
        pltpu.sync_copy(tvec_v, shared)
        sidx_v[pl.ds(0, 16)] = io * 0
        sidx_v[pl.ds(16, 16)] = io * 0 + 1
        pltpu.sync_copy(wp_hbm, w_v)
        pltpu.sync_copy(bp_hbm, b_v)
        idx_bcast.wait()

        ivec = idxb_v[...] * EMBED_DIM
        fidx_v[pl.ds(0, 16)] = ivec + io
        fidx_v[pl.ds(16, 16)] = ivec + jnp.minimum(io + 16, EMBED_DIM - 1)
        pltpu.async_copy(tflat_hbm.at[fidx_v], rowg_v, sem).wait()

        ra = rowg_v[pl.ds(0, 16)]
        rb = rowg_v[pl.ds(16, 16)]
        dwa = w_v[1, pl.ds(0, 16)] - w_v[0, pl.ds(0, 16)]
        dwb = w_v[1, pl.ds(16, 16)] - w_v[0, pl.ds(16, 16)]
        bg = b_v[...]
        bsign = jnp.where(io == 0, -1.0, jnp.where(io == 1, 1.0, 0.0))

        s = ra * dwa + rb * dwb + bg * bsign
        sprod_v[pl.ds(0, 16)] = s
        sprod_v[pl.ds(16, 16)] = -s

        pltpu.sync_copy(sprod_v, shared.at[sidx_v], add=True)
        pltpu.sync_copy(shared, tvec_v)

        t = tvec_v[...]
        a = jnp.maximum(t, 0.0)
        u = jnp.exp(-jnp.abs(t))
        x = 1.0 + u
        y = u * (6.0 + u) / (6.0 + 4.0 * u)
        y = y - 1.0 + x * jnp.exp(-y)
        y = y - 1.0 + x * jnp.exp(-y)
        out_v[0, pl.ds(0, 16)] = -(a + y)

        pltpu.sync_copy(out_v, out_hbm)


def kernel(indices, emb_table, W, b):
    tflat = emb_table.reshape(-1)
    wp = jnp.pad(W, ((0, 0), (0, 32 - EMBED_DIM)))
    bp = jnp.pad(b, (0, 16 - OUT_DIM))
    out = _sc_lookup_logsoftmax(indices.astype(jnp.int32), tflat, wp, bp)
    return out[:, :OUT_DIM]


# speedup vs baseline: 1.0063x; 1.0063x over previous
import functools

import jax
import jax.numpy as jnp
from jax import lax
from jax.experimental import pallas as pl
from jax.experimental.pallas import tpu as pltpu
from jax.experimental.pallas import tpu_sc as plsc

NUM_ELEMENTS = 1000000
EMBED_DIM = 20
OUT_DIM = 2


@functools.partial(
    pl.kernel,
    out_type=jax.ShapeDtypeStruct((1, 16), jnp.float32),
    mesh=plsc.VectorSubcoreMesh(core_axis_name="c", subcore_axis_name="s"),
    scratch_types=[
        pltpu.VMEM((16,), jnp.int32),
        pltpu.VMEM((16,), jnp.int32),
        pltpu.VMEM((32,), jnp.int32),
        pltpu.VMEM((32,), jnp.float32),
        pltpu.VMEM((2, 32), jnp.float32),
        pltpu.VMEM((16,), jnp.float32),
        pltpu.VMEM((32,), jnp.float32),
        pltpu.VMEM((32,), jnp.int32),
        pltpu.VMEM((16,), jnp.float32),
        pltpu.VMEM((1, 16), jnp.float32),
        pltpu.VMEM_SHARED((16,), jnp.float32),
        pltpu.SemaphoreType.DMA,
    ],
)
def _sc_lookup_logsoftmax(idx_hbm, tflat_hbm, wp_hbm, bp_hbm, out_hbm,
                          zidx_v, idxb_v, fidx_v, rowg_v, w_v, b_v,
                          sprod_v, sidx_v, tvec_v, out_v, shared, sem):
    cid = lax.axis_index("c")
    sid = lax.axis_index("s")

    @pl.when(jnp.logical_and(cid == 0, sid == 0))
    def _():
        io = lax.iota(jnp.int32, 16)
        fzero = jnp.zeros((16,), jnp.float32)

        zidx_v[...] = io * 0
        idx_bcast = pltpu.async_copy(idx_hbm.at[zidx_v], idxb_v, sem)

        tvec_v[...] = fzero
        pltpu.sync_copy(tvec_v, shared)
        sidx_v[pl.ds(0, 16)] = io * 0
        sidx_v[pl.ds(16, 16)] = io * 0 + 1
        pltpu.sync_copy(wp_hbm, w_v)
        pltpu.sync_copy(bp_hbm, b_v)
        idx_bcast.wait()

        ivec = idxb_v[...] * EMBED_DIM
        fidx_v[pl.ds(0, 16)] = ivec + io
        fidx_v[pl.ds(16, 16)] = ivec + jnp.minimum(io + 16, EMBED_DIM - 1)
        pltpu.async_copy(tflat_hbm.at[fidx_v], rowg_v, sem).wait()

        ra = rowg_v[pl.ds(0, 16)]
        rb = rowg_v[pl.ds(16, 16)]
        dwa = w_v[1, pl.ds(0, 16)] - w_v[0, pl.ds(0, 16)]
        dwb = w_v[1, pl.ds(16, 16)] - w_v[0, pl.ds(16, 16)]
        bg = b_v[...]
        bsign = jnp.where(io == 0, -1.0, jnp.where(io == 1, 1.0, 0.0))

        s = ra * dwa + rb * dwb + bg * bsign
        sprod_v[pl.ds(0, 16)] = s
        sprod_v[pl.ds(16, 16)] = -s

        pltpu.sync_copy(sprod_v, shared.at[sidx_v], add=True)
        pltpu.sync_copy(shared, tvec_v)

        t = tvec_v[...]
        a = jnp.maximum(t, 0.0)
        u = jnp.exp(-jnp.abs(t))
        x = 1.0 + u
        y = u * (6.0 + u) / (6.0 + 4.0 * u)
        y = y - 1.0 + x * jnp.exp(-y)
        y = y - 1.0 + x * jnp.exp(-y)
        out_v[0, pl.ds(0, 16)] = -(a + y)

        pltpu.sync_copy(out_v, out_hbm)


def kernel(indices, emb_table, W, b):
    tflat = emb_table.reshape(-1)
    wp = jnp.pad(W, ((0, 0), (0, 32 - EMBED_DIM)))
    bp = jnp.pad(b, (0, 16 - OUT_DIM))
    out = _sc_lookup_logsoftmax(indices.astype(jnp.int32), tflat, wp, bp)
    return out[:, :OUT_DIM]


# speedup vs baseline: 2.8151x; 2.7973x over previous
import functools

import jax
import jax.numpy as jnp
from jax import lax
from jax.experimental import pallas as pl
from jax.experimental.pallas import tpu as pltpu
from jax.experimental.pallas import tpu_sc as plsc

NUM_ELEMENTS = 1000000
EMBED_DIM = 20
OUT_DIM = 2


@functools.partial(
    pl.kernel,
    out_type=jax.ShapeDtypeStruct((1, 16), jnp.float32),
    mesh=plsc.VectorSubcoreMesh(core_axis_name="c", subcore_axis_name="s"),
    scratch_types=[
        pltpu.VMEM((16,), jnp.int32),
        pltpu.VMEM((16,), jnp.int32),
        pltpu.VMEM((EMBED_DIM,), jnp.float32),
        pltpu.VMEM((2, 32), jnp.float32),
        pltpu.VMEM((16,), jnp.float32),
        pltpu.VMEM((32,), jnp.float32),
        pltpu.VMEM((32,), jnp.int32),
        pltpu.VMEM((16,), jnp.float32),
        pltpu.VMEM((1, 16), jnp.float32),
        pltpu.VMEM_SHARED((16,), jnp.float32),
        pltpu.SemaphoreType.DMA,
    ],
)
def _sc_lookup_logsoftmax(idx_hbm, table_hbm, wp_hbm, bp_hbm, out_hbm,
                          zidx_v, idxb_v, row_v, w_v, b_v,
                          sprod_v, sidx_v, tvec_v, out_v, shared, sem):
    cid = lax.axis_index("c")
    sid = lax.axis_index("s")

    @pl.when(jnp.logical_and(cid == 0, sid == 0))
    def _():
        io = lax.iota(jnp.int32, 16)
        fzero = jnp.zeros((16,), jnp.float32)

        zidx_v[...] = io * 0
        pltpu.async_copy(idx_hbm.at[zidx_v], idxb_v, sem).wait()
        i = idxb_v[...][0]
        row_fetch = pltpu.async_copy(table_hbm.at[i], row_v, sem)

        tvec_v[...] = fzero
        pltpu.sync_copy(tvec_v, shared)
        sidx_v[pl.ds(0, 16)] = io * 0
        sidx_v[pl.ds(16, 16)] = io * 0 + 1
        pltpu.sync_copy(wp_hbm, w_v)
        pltpu.sync_copy(bp_hbm, b_v)
        row_fetch.wait()

        ra = row_v[pl.ds(0, 16)]
        rb = row_v[pl.ds(4, 16)]
        dwa = w_v[1, pl.ds(0, 16)] - w_v[0, pl.ds(0, 16)]
        dwb = w_v[1, pl.ds(16, 16)] - w_v[0, pl.ds(16, 16)]
        bg = b_v[...]
        bsign = jnp.where(io == 0, -1.0, jnp.where(io == 1, 1.0, 0.0))

        s = ra * dwa + rb * dwb + bg * bsign
        sprod_v[pl.ds(0, 16)] = s
        sprod_v[pl.ds(16, 16)] = -s

        pltpu.sync_copy(sprod_v, shared.at[sidx_v], add=True)
        pltpu.sync_copy(shared, tvec_v)

        t = tvec_v[...]
        a = jnp.maximum(t, 0.0)
        u = jnp.exp(-jnp.abs(t))
        x = 1.0 + u
        y = u * (6.0 + u) / (6.0 + 4.0 * u)
        y = y - 1.0 + x * jnp.exp(-y)
        y = y - 1.0 + x * jnp.exp(-y)
        out_v[0, pl.ds(0, 16)] = -(a + y)

        pltpu.sync_copy(out_v, out_hbm)


def kernel(indices, emb_table, W, b):
    wp = jnp.concatenate(
        [W[:, :16], jnp.zeros((OUT_DIM, 12), W.dtype), W[:, 16:]], axis=1)
    bp = jnp.pad(b, (0, 16 - OUT_DIM))
    out = _sc_lookup_logsoftmax(indices.astype(jnp.int32), emb_table, wp, bp)
    return out[:, :OUT_DIM]


# speedup vs baseline: 36.7581x; 13.0577x over previous
import functools

import jax
import jax.numpy as jnp
from jax import lax
from jax.experimental import pallas as pl
from jax.experimental.pallas import tpu as pltpu
from jax.experimental.pallas import tpu_sc as plsc

NUM_ELEMENTS = 1000000
EMBED_DIM = 20
OUT_DIM = 2


@functools.partial(
    pl.kernel,
    out_type=jax.ShapeDtypeStruct((1, 16), jnp.float32),
    mesh=plsc.VectorSubcoreMesh(core_axis_name="c", subcore_axis_name="s"),
    scratch_types=[
        pltpu.VMEM((16,), jnp.int32),
        pltpu.VMEM((16,), jnp.int32),
        pltpu.VMEM((EMBED_DIM, 128), jnp.float32),
        pltpu.VMEM((2, 32), jnp.float32),
        pltpu.VMEM((16,), jnp.float32),
        pltpu.VMEM((32,), jnp.float32),
        pltpu.VMEM((32,), jnp.int32),
        pltpu.VMEM((16,), jnp.float32),
        pltpu.VMEM((1, 16), jnp.float32),
        pltpu.VMEM_SHARED((16,), jnp.float32),
        pltpu.SemaphoreType.DMA,
    ],
)
def _sc_lookup_logsoftmax(idx_hbm, tablet_hbm, wp_hbm, bp_hbm, out_hbm,
                          zidx_v, idxb_v, colblk_v,
                          w_v, b_v, sprod_v, sidx_v, tvec_v, out_v,
                          shared_d, sem):
    cid = lax.axis_index("c")
    sid = lax.axis_index("s")

    @pl.when(jnp.logical_and(cid == 0, sid == 0))
    def _():
        io = lax.iota(jnp.int32, 16)
        fzero = jnp.zeros((16,), jnp.float32)

        zidx_v[...] = io * 0
        idx_fetch = pltpu.async_copy(idx_hbm.at[zidx_v], idxb_v, sem)

        tvec_v[...] = fzero
        pltpu.sync_copy(tvec_v, shared_d)
        sidx_v[pl.ds(0, 16)] = io * 0
        sidx_v[pl.ds(16, 16)] = io * 0 + 1
        pltpu.sync_copy(wp_hbm, w_v)
        pltpu.sync_copy(bp_hbm, b_v)

        idx_fetch.wait()
        i = idxb_v[...][0]

        r = lax.rem(i, 128)
        base = pl.multiple_of(i - r, 128)
        blk_fetch = pltpu.async_copy(
            tablet_hbm.at[:, pl.ds(base, 128)], colblk_v, sem)

        dwa = w_v[1, pl.ds(0, 16)] - w_v[0, pl.ds(0, 16)]
        dwb = w_v[1, pl.ds(16, 16)] - w_v[0, pl.ds(16, 16)]
        bg = b_v[...]
        bsign = jnp.where(io == 0, -1.0, jnp.where(io == 1, 1.0, 0.0))
        rdiv = lax.div(r, 16)
        blk_fetch.wait()

        for c in range(8):
            @pl.when(rdiv == c)
            def _(c=c):
                wsum = fzero
                for j in range(EMBED_DIM):
                    dwj = dwa[j] if j < 16 else dwb[j - 16]
                    wsum = wsum + colblk_v[j, pl.ds(16 * c, 16)] * dwj
                oh = jnp.where(io == r - 16 * c, 1.0, 0.0)
                s = wsum * oh + bg * bsign
                sprod_v[pl.ds(0, 16)] = s
                sprod_v[pl.ds(16, 16)] = -s

        pltpu.sync_copy(sprod_v, shared_d.at[sidx_v], add=True)
        pltpu.sync_copy(shared_d, tvec_v)

        t = tvec_v[...]
        a = jnp.maximum(t, 0.0)
        u = jnp.exp(-jnp.abs(t))
        x = 1.0 + u
        y = u * (6.0 + u) / (6.0 + 4.0 * u)
        y = y - 1.0 + x * jnp.exp(-y)
        y = y - 1.0 + x * jnp.exp(-y)
        out_v[0, pl.ds(0, 16)] = -(a + y)

        pltpu.sync_copy(out_v, out_hbm)


def kernel(indices, emb_table, W, b):
    table_t = emb_table.T
    wp = jnp.pad(W, ((0, 0), (0, 32 - EMBED_DIM)))
    bp = jnp.pad(b, (0, 16 - OUT_DIM))
    out = _sc_lookup_logsoftmax(indices.astype(jnp.int32), table_t, wp, bp)
    return out[:, :OUT_DIM]


# speedup vs baseline: 40.1512x; 1.0923x over previous
import functools

import jax
import jax.numpy as jnp
from jax import lax
from jax.experimental import pallas as pl
from jax.experimental.pallas import tpu as pltpu
from jax.experimental.pallas import tpu_sc as plsc

NUM_ELEMENTS = 1000000
EMBED_DIM = 20
OUT_DIM = 2


@functools.partial(
    pl.kernel,
    out_type=jax.ShapeDtypeStruct((1, 16), jnp.float32),
    mesh=plsc.VectorSubcoreMesh(core_axis_name="c", subcore_axis_name="s",
                                num_cores=1),
    scratch_types=[
        pltpu.VMEM((16,), jnp.int32),
        pltpu.VMEM((16,), jnp.int32),
        pltpu.VMEM((16,), jnp.int32),
        pltpu.VMEM((16,), jnp.float32),
        pltpu.VMEM((EMBED_DIM, 128), jnp.float32),
        pltpu.VMEM((2, EMBED_DIM), jnp.float32),
        pltpu.VMEM((32,), jnp.float32),
        pltpu.VMEM((32,), jnp.int32),
        pltpu.VMEM((16,), jnp.float32),
        pltpu.VMEM((1, 16), jnp.float32),
        pltpu.VMEM_SHARED((16,), jnp.float32),
        pltpu.SemaphoreType.DMA,
        pltpu.SemaphoreType.DMA,
    ],
)
def _sc_lookup_logsoftmax(idx_hbm, tablet_hbm, w_hbm, b_hbm, out_hbm,
                          zidx_v, idxb_v, bidx_v, bb_v, colblk_v, w_v,
                          sprod_v, sidx_v, tvec_v, out_v,
                          shared_d, sem, sem2):
    sid = lax.axis_index("s")

    @pl.when(sid == 0)
    def _():
        io = lax.iota(jnp.int32, 16)
        fzero = jnp.zeros((16,), jnp.float32)

        zidx_v[...] = io * 0
        bidx_v[...] = jnp.minimum(io, 1)
        idx_fetch = pltpu.async_copy(idx_hbm.at[zidx_v], idxb_v, sem)
        b_fetch = pltpu.async_copy(b_hbm.at[bidx_v], bb_v, sem2)

        tvec_v[...] = fzero
        pltpu.sync_copy(tvec_v, shared_d)
        sidx_v[pl.ds(0, 16)] = io * 0
        sidx_v[pl.ds(16, 16)] = io * 0 + 1
        pltpu.sync_copy(w_hbm, w_v)

        idx_fetch.wait()
        i = idxb_v[...][0]

        r = lax.rem(i, 128)
        base = pl.multiple_of(i - r, 128)
        blk_fetch = pltpu.async_copy(
            tablet_hbm.at[:, pl.ds(base, 128)], colblk_v, sem)

        dwa = w_v[1, pl.ds(0, 16)] - w_v[0, pl.ds(0, 16)]
        dwb_ov = w_v[1, pl.ds(4, 16)] - w_v[0, pl.ds(4, 16)]
        dwb = jnp.where(io >= 12, dwb_ov, 0.0)
        b_fetch.wait()
        bb = bb_v[...]
        bsign = jnp.where(io == 0, -1.0, jnp.where(io == 1, 1.0, 0.0))
        bterm = bb * bsign
        rdiv = lax.div(r, 16)
        blk_fetch.wait()

        for c in range(8):
            @pl.when(rdiv == c)
            def _(c=c):
                wsum = fzero
                for j in range(EMBED_DIM):
                    dwj = dwa[j] if j < 16 else dwb[j - 4]
                    wsum = wsum + colblk_v[j, pl.ds(16 * c, 16)] * dwj
                oh = jnp.where(io == r - 16 * c, 1.0, 0.0)
                s = wsum * oh + bterm
                sprod_v[pl.ds(0, 16)] = s
                sprod_v[pl.ds(16, 16)] = -s

        pltpu.sync_copy(sprod_v, shared_d.at[sidx_v], add=True)
        pltpu.sync_copy(shared_d, tvec_v)

        t = tvec_v[...]
        a = jnp.maximum(t, 0.0)
        u = jnp.exp(-jnp.abs(t))
        x = 1.0 + u
        y = u * (6.0 + u) / (6.0 + 4.0 * u)
        y = y - 1.0 + x * jnp.exp(-y)
        y = y - 1.0 + x * jnp.exp(-y)
        out_v[0, pl.ds(0, 16)] = -(a + y)

        pltpu.sync_copy(out_v, out_hbm)


def kernel(indices, emb_table, W, b):
    table_t = emb_table.T
    out = _sc_lookup_logsoftmax(indices.astype(jnp.int32), table_t, W, b)
    return out[:, :OUT_DIM]


# speedup vs baseline: 41.6388x; 1.0370x over previous
import functools

import jax
import jax.numpy as jnp
from jax import lax
from jax.experimental import pallas as pl
from jax.experimental.pallas import tpu as pltpu
from jax.experimental.pallas import tpu_sc as plsc

NUM_ELEMENTS = 1000000
EMBED_DIM = 20
OUT_DIM = 2


@functools.partial(
    pl.kernel,
    out_type=jax.ShapeDtypeStruct((1, 16), jnp.float32),
    mesh=plsc.VectorSubcoreMesh(core_axis_name="c", subcore_axis_name="s",
                                num_cores=1),
    scratch_types=[
        pltpu.VMEM((16,), jnp.int32),
        pltpu.VMEM((16,), jnp.int32),
        pltpu.VMEM((16,), jnp.int32),
        pltpu.VMEM((16,), jnp.float32),
        pltpu.VMEM((EMBED_DIM, 128), jnp.float32),
        pltpu.VMEM((2, EMBED_DIM), jnp.float32),
        pltpu.VMEM((32,), jnp.float32),
        pltpu.VMEM((32,), jnp.int32),
        pltpu.VMEM((16,), jnp.float32),
        pltpu.VMEM((1, 16), jnp.float32),
        pltpu.VMEM_SHARED((16,), jnp.float32),
        pltpu.SemaphoreType.DMA,
        pltpu.SemaphoreType.DMA,
    ],
)
def _sc_lookup_logsoftmax(idx_hbm, tablet_hbm, w_hbm, b_hbm, out_hbm,
                          zidx_v, idxb_v, bidx_v, bb_v, colblk_v, w_v,
                          sprod_v, sidx_v, tvec_v, out_v,
                          shared_d, sem, sem2):
    sid = lax.axis_index("s")

    @pl.when(sid == 0)
    def _():
        io = lax.iota(jnp.int32, 16)
        fzero = jnp.zeros((16,), jnp.float32)

        zidx_v[...] = io * 0
        bidx_v[...] = jnp.minimum(io, 1)
        idx_fetch = pltpu.async_copy(idx_hbm.at[zidx_v], idxb_v, sem)
        b_fetch = pltpu.async_copy(b_hbm.at[bidx_v], bb_v, sem2)

        tvec_v[...] = fzero
        pltpu.sync_copy(tvec_v, shared_d)
        sidx_v[pl.ds(0, 16)] = io * 0
        sidx_v[pl.ds(16, 16)] = io * 0 + 1
        pltpu.sync_copy(w_hbm, w_v)

        idx_fetch.wait()
        i = idxb_v[...][0]

        r = lax.rem(i, 128)
        base = pl.multiple_of(i - r, 128)
        blk_fetch = pltpu.async_copy(
            tablet_hbm.at[:, pl.ds(base, 128)], colblk_v, sem)

        dwa = w_v[1, pl.ds(0, 16)] - w_v[0, pl.ds(0, 16)]
        dwb_ov = w_v[1, pl.ds(4, 16)] - w_v[0, pl.ds(4, 16)]
        dwb = jnp.where(io >= 12, dwb_ov, 0.0)
        b_fetch.wait()
        bb = bb_v[...]
        bsign = jnp.where(io == 0, -1.0, jnp.where(io == 1, 1.0, 0.0))
        bterm = bb * bsign
        roff = pl.multiple_of(r - lax.rem(r, 16), 16)
        blk_fetch.wait()

        wsum = fzero
        for j in range(EMBED_DIM):
            dwj = dwa[j] if j < 16 else dwb[j - 4]
            wsum = wsum + colblk_v[j, pl.ds(roff, 16)] * dwj
        oh = jnp.where(io == r - roff, 1.0, 0.0)
        s = wsum * oh + bterm
        sprod_v[pl.ds(0, 16)] = s
        sprod_v[pl.ds(16, 16)] = -s

        pltpu.sync_copy(sprod_v, shared_d.at[sidx_v], add=True)
        pltpu.sync_copy(shared_d, tvec_v)

        t = tvec_v[...]
        a = jnp.maximum(t, 0.0)
        u = jnp.exp(-jnp.abs(t))
        x = 1.0 + u
        y = u * (6.0 + u) / (6.0 + 4.0 * u)
        y = y - 1.0 + x * jnp.exp(-y)
        y = y - 1.0 + x * jnp.exp(-y)
        out_v[0, pl.ds(0, 16)] = -(a + y)

        pltpu.sync_copy(out_v, out_hbm)


def kernel(indices, emb_table, W, b):
    table_t = emb_table.T
    out = _sc_lookup_logsoftmax(indices.astype(jnp.int32), table_t, W, b)
    return out[:, :OUT_DIM]
